# baseline (speedup 1.0000x reference)
import functools
import jax
import jax.numpy as jnp
from jax.experimental import pallas as pl
from jax.experimental.pallas import tpu as pltpu

EPS = 1e-5
_ISQ = float(1.0 / (1.0 + EPS) ** 0.5)


def _dot(a, b):
    return jax.lax.dot_general(a, b, (((1,), (0,)), ((), ())),
                               preferred_element_type=jnp.float32,
                               precision=jax.lax.Precision.HIGHEST)


def _layer_body(agg_ref, x_ref, deg_ref, W_ref, b_ref, Wr_ref, br_ref,
                g_ref, be_ref, o_ref):
    agg = _dot(agg_ref[...], W_ref[...]) + deg_ref[...] * b_ref[...]
    res = _dot(x_ref[...], Wr_ref[...]) + br_ref[...]
    new = jnp.maximum(agg, 0.0) + jnp.maximum(res, 0.0)
    o_ref[...] = g_ref[...] * new * _ISQ + be_ref[...]


def _layer_tc(aggX, x, deg, W, b, Wr, br, gamma, beta, blk=2000):
    N, Fin = x.shape
    H = W.shape[1]
    Fa = aggX.shape[1]
    grid = (N + blk - 1) // blk
    row = lambda i: (i, 0)
    zero = lambda i: (0, 0)
    return pl.pallas_call(
        _layer_body,
        grid=(grid,),
        in_specs=[
            pl.BlockSpec((blk, Fa), row),
            pl.BlockSpec((blk, Fin), row),
            pl.BlockSpec((blk, 1), row),
            pl.BlockSpec((Fa, H), zero),
            pl.BlockSpec((1, H), zero),
            pl.BlockSpec((Fin, H), zero),
            pl.BlockSpec((1, H), zero),
            pl.BlockSpec((1, H), zero),
            pl.BlockSpec((1, H), zero),
        ],
        out_specs=pl.BlockSpec((blk, H), row),
        out_shape=jax.ShapeDtypeStruct((N, H), jnp.float32),
    )(aggX, x, deg, W.reshape(Fa, H), b.reshape(1, H), Wr, br.reshape(1, H),
      gamma.reshape(1, H), beta.reshape(1, H))


def _layer2_body(agg_ref, x_ref, deg_ref, W_ref, b_ref, Wr_ref, br_ref,
                 g_ref, be_ref, aw_ref, ab_ref, h_ref, y_ref):
    agg = _dot(agg_ref[...], W_ref[...]) + deg_ref[...] * b_ref[...]
    res = _dot(x_ref[...], Wr_ref[...]) + br_ref[...]
    new = jnp.maximum(agg, 0.0) + jnp.maximum(res, 0.0)
    h = g_ref[...] * new * _ISQ + be_ref[...]
    h_ref[...] = h
    z = jnp.sum(h * aw_ref[...], axis=1, keepdims=True) + ab_ref[...]
    w = jax.nn.sigmoid(z)
    y_ref[...] = h * w


def _layer2_tc(aggH, h1, deg, W, b, Wr, br, gamma, beta, atom_w, atom_b,
               blk=2000):
    N, H = h1.shape
    grid = (N + blk - 1) // blk
    row = lambda i: (i, 0)
    zero = lambda i: (0, 0)
    return pl.pallas_call(
        _layer2_body,
        grid=(grid,),
        in_specs=[
            pl.BlockSpec((blk, H), row),
            pl.BlockSpec((blk, H), row),
            pl.BlockSpec((blk, 1), row),
            pl.BlockSpec((H, H), zero),
            pl.BlockSpec((1, H), zero),
            pl.BlockSpec((H, H), zero),
            pl.BlockSpec((1, H), zero),
            pl.BlockSpec((1, H), zero),
            pl.BlockSpec((1, H), zero),
            pl.BlockSpec((1, H), zero),
            pl.BlockSpec((1, 1), zero),
        ],
        out_specs=[pl.BlockSpec((blk, H), row), pl.BlockSpec((blk, H), row)],
        out_shape=[jax.ShapeDtypeStruct((N, H), jnp.float32),
                   jax.ShapeDtypeStruct((N, H), jnp.float32)],
    )(aggH, h1, deg, W, b.reshape(1, H), Wr, br.reshape(1, H),
      gamma.reshape(1, H), beta.reshape(1, H), atom_w.reshape(1, H),
      atom_b.reshape(1, 1))


def _pred_body(hs_ref, hm_ref, W1_ref, b1_ref, g_ref, be_ref, W2_ref, b2_ref,
               o_ref):
    hm = hm_ref[...]
    hm = jnp.where(jnp.isfinite(hm), hm, 0.0)
    g = _dot(hs_ref[...], W1_ref[0]) + _dot(hm, W1_ref[1]) + b1_ref[...]
    g = jnp.maximum(g, 0.0)
    g = g_ref[...] * g * _ISQ + be_ref[...]
    o_ref[...] = _dot(g, W2_ref[...]) + b2_ref[...]


def _pred_tc(hsum, hmax, p_W1, p_b1, p_gamma, p_beta, p_W2, p_b2):
    B, H = hsum.shape
    P = p_W1.shape[1]
    T = p_W2.shape[1]
    zero2 = lambda: pl.BlockSpec(None, None)
    return pl.pallas_call(
        _pred_body,
        in_specs=[pl.BlockSpec(hsum.shape, None),
                  pl.BlockSpec(hmax.shape, None),
                  pl.BlockSpec((2, H, P), None),
                  pl.BlockSpec((1, P), None),
                  pl.BlockSpec((1, P), None),
                  pl.BlockSpec((1, P), None),
                  pl.BlockSpec((P, T), None),
                  pl.BlockSpec((1, T), None)],
        out_specs=pl.BlockSpec((B, T), None),
        out_shape=jax.ShapeDtypeStruct((B, T), jnp.float32),
    )(hsum, hmax, p_W1.reshape(2, H, P), p_b1.reshape(1, P),
      p_gamma.reshape(1, P), p_beta.reshape(1, P), p_W2, p_b2.reshape(1, T))


def _final_body(c_ref, s1_ref, s2_ref, Wc_ref, Ws1_ref, Ws2_ref, b_ref,
                g_ref, be_ref, ow_ref, ob_ref, o_ref):
    h = (_dot(c_ref[...], Wc_ref[...]) + _dot(s1_ref[...], Ws1_ref[...])
         + _dot(s2_ref[...], Ws2_ref[...]) + b_ref[...])
    h = g_ref[...] * h * _ISQ + be_ref[...]
    h = jnp.maximum(h, 0.0)
    z = jnp.sum(h * ow_ref[...], axis=1, keepdims=True) + ob_ref[...]
    o_ref[...] = jax.nn.sigmoid(z)


def _final_tc(core, sub1, sub2, fc_W, fc_b, fc_gamma, fc_beta, out_W, out_b):
    B, Tc = core.shape
    T1 = sub1.shape[1]
    F = fc_W.shape[1]
    return pl.pallas_call(
        _final_body,
        in_specs=[pl.BlockSpec(core.shape, None),
                  pl.BlockSpec(sub1.shape, None),
                  pl.BlockSpec(sub2.shape, None),
                  pl.BlockSpec((Tc, F), None),
                  pl.BlockSpec((T1, F), None),
                  pl.BlockSpec((T1, F), None),
                  pl.BlockSpec((1, F), None),
                  pl.BlockSpec((1, F), None),
                  pl.BlockSpec((1, F), None),
                  pl.BlockSpec((1, F), None),
                  pl.BlockSpec((1, 1), None)],
        out_specs=pl.BlockSpec((B, 1), None),
        out_shape=jax.ShapeDtypeStruct((B, 1), jnp.float32),
    )(core, sub1, sub2, fc_W[:Tc], fc_W[Tc:Tc + T1], fc_W[Tc + T1:],
      fc_b.reshape(1, F), fc_gamma.reshape(1, F), fc_beta.reshape(1, F),
      out_W.reshape(1, F), out_b.reshape(1, 1))


def _edge_agg(V, src, dst, N):
    return jnp.zeros((N, V.shape[1]), jnp.float32).at[dst].add(V[src])


def _gcn_predictor(x, edge_index, gids, p, B):
    src, dst = edge_index[0], edge_index[1]
    N = x.shape[0]
    x_aug = jnp.concatenate([x, jnp.ones((N, 1), jnp.float32)], axis=1)
    A1 = _edge_agg(x_aug, src, dst, N)
    deg = A1[:, -1:]
    lp1, lp2 = p['layers']
    h1 = _layer_tc(A1[:, :-1], x, deg, lp1['W'], lp1['b'], lp1['Wr'],
                   lp1['br'], lp1['gamma'], lp1['beta'])
    A2 = _edge_agg(h1, src, dst, N)
    h2, y = _layer2_tc(A2, h1, deg, lp2['W'], lp2['b'], lp2['Wr'], lp2['br'],
                       lp2['gamma'], lp2['beta'], p['atom_w'], p['atom_b'])
    hsum = jax.ops.segment_sum(y, gids, num_segments=B)
    hmax = jax.ops.segment_max(h2, gids, num_segments=B)
    return _pred_tc(hsum, hmax, p['p_W1'], p['p_b1'], p['p_gamma'],
                    p['p_beta'], p['p_W2'], p['p_b2'])


def kernel(x_core, x_sub1, x_sub2, edge_index_core, edge_index_sub1,
           edge_index_sub2, graph_ids_core, graph_ids_sub1, graph_ids_sub2,
           params):
    B = 256
    core = _gcn_predictor(x_core, edge_index_core, graph_ids_core,
                          params['core'], B)
    sub1 = _gcn_predictor(x_sub1, edge_index_sub1, graph_ids_sub1,
                          params['sub'], B)
    sub2 = _gcn_predictor(x_sub2, edge_index_sub2, graph_ids_sub2,
                          params['sub'], B)
    out = _final_tc(core, sub1, sub2, params['fc_W'], params['fc_b'],
                    params['fc_gamma'], params['fc_beta'], params['out_W'],
                    params['out_b'])
    return out.reshape(-1)


# speedup vs baseline: 2.5583x; 2.5583x over previous
import functools
import numpy as np
import jax
import jax.numpy as jnp
from jax import lax
from jax.experimental import pallas as pl
from jax.experimental.pallas import tpu as pltpu
from jax.experimental.pallas import tpu_sc as plsc

EPS = 1e-5
_SQ = float(np.sqrt(np.float32(1.0 + EPS)))
N = 10000
E = 160000
B = 256


def _dot(a, b):
    return jax.lax.dot_general(a, b, (((1,), (0,)), ((), ())),
                               preferred_element_type=jnp.float32)


def _edge_agg_body(F, v_hbm, src_hbm, dst_hbm, out_hbm,
                   stg, stg_t, si, di, si_t, di_t, zv, agg, sem):
    EPW = E // 32
    NCH = EPW // 128
    TAIL = EPW - NCH * 128
    RPT = 632
    LROWS = N - 15 * RPT
    c = lax.axis_index("c")
    s = lax.axis_index("s")

    def zrow(r, carry):
        for j in range(F // 16):
            zv[r, pl.ds(j * 16, 16)] = jnp.zeros((16,), jnp.float32)
        return carry
    lax.fori_loop(0, 128, zrow, 0)

    r0 = s * RPT

    def zero_rows(rows):
        nzc = rows // 128
        for j in range(nzc):
            pltpu.sync_copy(zv, agg.at[pl.ds(r0 + j * 128, 128)])
        rem = rows - nzc * 128
        if rem:
            pltpu.sync_copy(zv.at[pl.ds(0, rem)],
                            agg.at[pl.ds(r0 + nzc * 128, rem)])

    @pl.when(s < 15)
    def _():
        zero_rows(RPT)

    @pl.when(s == 15)
    def _():
        zero_rows(LROWS)
    plsc.subcore_barrier()

    base = (c * 16 + s) * EPW

    def chunk(off, K, stg_, si_, di_):
        pltpu.sync_copy(src_hbm.at[pl.ds(off, K)], si_)
        pltpu.sync_copy(dst_hbm.at[pl.ds(off, K)], di_)
        pltpu.async_copy(v_hbm.at[si_], stg_, sem).wait()
        pltpu.sync_copy(stg_, agg.at[di_], add=True)

    def loop(ci, carry):
        chunk(base + ci * 128, 128, stg, si, di)
        return carry
    lax.fori_loop(0, NCH, loop, 0)
    if TAIL:
        chunk(base + NCH * 128, TAIL, stg_t, si_t, di_t)
    plsc.subcore_barrier()

    @pl.when(s < 15)
    def _():
        pltpu.sync_copy(agg.at[pl.ds(r0, RPT)], out_hbm.at[c, pl.ds(r0, RPT)])

    @pl.when(s == 15)
    def _():
        pltpu.sync_copy(agg.at[pl.ds(r0, LROWS)],
                        out_hbm.at[c, pl.ds(r0, LROWS)])


@functools.lru_cache(maxsize=None)
def _make_edge_agg(F):
    TAIL = (E // 32) % 128
    mesh = plsc.VectorSubcoreMesh(core_axis_name="c", subcore_axis_name="s")
    return pl.kernel(
        functools.partial(_edge_agg_body, F),
        out_type=jax.ShapeDtypeStruct((2, N, F), jnp.float32),
        mesh=mesh,
        compiler_params=pltpu.CompilerParams(use_tc_tiling_on_sc=False),
        scratch_types=[
            pltpu.VMEM((128, F), jnp.float32),
            pltpu.VMEM((TAIL, F), jnp.float32),
            pltpu.VMEM((128,), jnp.int32),
            pltpu.VMEM((128,), jnp.int32),
            pltpu.VMEM((TAIL,), jnp.int32),
            pltpu.VMEM((TAIL,), jnp.int32),
            pltpu.VMEM((128, F), jnp.float32),
            pltpu.VMEM_SHARED((N, F), jnp.float32),
            pltpu.SemaphoreType.DMA,
        ])


def _pre1_body(halves, x_ref, W_ref, b_ref, *o_refs):
    h = _dot(x_ref[...], W_ref[...]) + b_ref[...]
    off = 0
    for i, hw in enumerate(halves):
        o_refs[i][...] = h[:, off:off + hw]
        off += hw


def _pre1_tc(x, W, b, halves, blk=2000):
    H = W.shape[1]
    grid = N // blk
    row = lambda i: (i, 0)
    zero = lambda i: (0, 0)
    return pl.pallas_call(
        functools.partial(_pre1_body, tuple(halves)),
        grid=(grid,),
        in_specs=[pl.BlockSpec((blk, 32), row),
                  pl.BlockSpec((32, H), zero),
                  pl.BlockSpec((1, H), zero)],
        out_specs=[pl.BlockSpec((blk, hw), row) for hw in halves],
        out_shape=[jax.ShapeDtypeStruct((N, hw), jnp.float32)
                   for hw in halves],
    )(x, W, b.reshape(1, H))


def _mid_body(halves, nh, *refs):
    a_refs = refs[:nh]
    (x_ref, Wr_ref, br_ref, g_ref, be_ref, W2_ref, b2_ref) = refs[nh:nh + 7]
    o_refs = refs[nh + 7:]
    agg = jnp.concatenate([a[0] + a[1] for a in a_refs], axis=1) \
        if nh > 1 else a_refs[0][0] + a_refs[0][1]
    res = _dot(x_ref[...], Wr_ref[...]) + br_ref[...]
    new = jnp.maximum(agg, 0.0) + jnp.maximum(res, 0.0)
    h1 = g_ref[...] * new / _SQ + be_ref[...]
    o_refs[0][...] = h1
    hpre2 = _dot(h1, W2_ref[...]) + b2_ref[...]
    off = 0
    for i, hw in enumerate(halves):
        o_refs[1 + i][...] = hpre2[:, off:off + hw]
        off += hw


def _mid_tc(A1p, x, Wr, br, gamma, beta, W2, b2, halves, blk=2000):
    H = Wr.shape[1]
    grid = N // blk
    row = lambda i: (i, 0)
    row3 = lambda i: (0, i, 0)
    zero = lambda i: (0, 0)
    nh = len(halves)
    outs = pl.pallas_call(
        functools.partial(_mid_body, tuple(halves), nh),
        grid=(grid,),
        in_specs=[pl.BlockSpec((2, blk, hw), row3) for hw in halves]
        + [pl.BlockSpec((blk, 32), row),
           pl.BlockSpec((32, H), zero),
           pl.BlockSpec((1, H), zero),
           pl.BlockSpec((1, H), zero),
           pl.BlockSpec((1, H), zero),
           pl.BlockSpec((H, H), zero),
           pl.BlockSpec((1, H), zero)],
        out_specs=[pl.BlockSpec((blk, H), row)]
        + [pl.BlockSpec((blk, hw), row) for hw in halves],
        out_shape=[jax.ShapeDtypeStruct((N, H), jnp.float32)]
        + [jax.ShapeDtypeStruct((N, hw), jnp.float32) for hw in halves],
    )(*A1p, x, Wr, br.reshape(1, H), gamma.reshape(1, H), beta.reshape(1, H),
      W2, b2.reshape(1, H))
    return outs[0], outs[1:]


def _post_body(nh, *refs):
    a_refs = refs[:nh]
    (h1_ref, Wr_ref, br_ref, g_ref, be_ref, aw_ref, ab_ref) = refs[nh:nh + 7]
    h2_ref, y_ref = refs[nh + 7:]
    agg = jnp.concatenate([a[0] + a[1] for a in a_refs], axis=1) \
        if nh > 1 else a_refs[0][0] + a_refs[0][1]
    res = _dot(h1_ref[...], Wr_ref[...]) + br_ref[...]
    new = jnp.maximum(agg, 0.0) + jnp.maximum(res, 0.0)
    h2 = g_ref[...] * new / _SQ + be_ref[...]
    h2_ref[...] = h2
    z = _dot(h2, aw_ref[...]) + ab_ref[...]
    y_ref[...] = h2 * jax.nn.sigmoid(z)


def _post_tc(A2p, h1, Wr, br, gamma, beta, atom_w, atom_b, halves, blk=2000):
    H = Wr.shape[1]
    grid = N // blk
    row = lambda i: (i, 0)
    row3 = lambda i: (0, i, 0)
    zero = lambda i: (0, 0)
    nh = len(halves)
    return pl.pallas_call(
        functools.partial(_post_body, nh),
        grid=(grid,),
        in_specs=[pl.BlockSpec((2, blk, hw), row3) for hw in halves]
        + [pl.BlockSpec((blk, H), row),
           pl.BlockSpec((H, H), zero),
           pl.BlockSpec((1, H), zero),
           pl.BlockSpec((1, H), zero),
           pl.BlockSpec((1, H), zero),
           pl.BlockSpec((H, 1), zero),
           pl.BlockSpec((1, 1), zero)],
        out_specs=[pl.BlockSpec((blk, H), row), pl.BlockSpec((blk, H), row)],
        out_shape=[jax.ShapeDtypeStruct((N, H), jnp.float32),
                   jax.ShapeDtypeStruct((N, H), jnp.float32)],
    )(*A2p, h1, Wr, br.reshape(1, H), gamma.reshape(1, H),
      beta.reshape(1, H), atom_w, atom_b.reshape(1, 1))


def _pred_body(cat_ref, W1_ref, b1_ref, g_ref, be_ref, W2_ref, b2_ref,
               o_ref):
    g = _dot(cat_ref[...], W1_ref[...]) + b1_ref[...]
    g = jnp.maximum(g, 0.0)
    g = g_ref[...] * g / _SQ + be_ref[...]
    o_ref[...] = _dot(g, W2_ref[...]) + b2_ref[...]


def _pred_tc(cat, p_W1, p_b1, p_gamma, p_beta, p_W2, p_b2):
    H2 = cat.shape[1]
    P = p_W1.shape[1]
    T = p_W2.shape[1]
    return pl.pallas_call(
        _pred_body,
        in_specs=[pl.BlockSpec(cat.shape, None),
                  pl.BlockSpec((H2, P), None),
                  pl.BlockSpec((1, P), None),
                  pl.BlockSpec((1, P), None),
                  pl.BlockSpec((1, P), None),
                  pl.BlockSpec((P, T), None),
                  pl.BlockSpec((1, T), None)],
        out_specs=pl.BlockSpec((B, T), None),
        out_shape=jax.ShapeDtypeStruct((B, T), jnp.float32),
    )(cat, p_W1, p_b1.reshape(1, P), p_gamma.reshape(1, P),
      p_beta.reshape(1, P), p_W2, p_b2.reshape(1, T))


def _final_body(o_ref_in, W_ref, b_ref, g_ref, be_ref, ow_ref, ob_ref,
                o_ref):
    h = _dot(o_ref_in[...], W_ref[...]) + b_ref[...]
    h = g_ref[...] * h / _SQ + be_ref[...]
    h = jnp.maximum(h, 0.0)
    z = _dot(h, ow_ref[...]) + ob_ref[...]
    o_ref[...] = jax.nn.sigmoid(z)


def _final_tc(cat, fc_W, fc_b, fc_gamma, fc_beta, out_W, out_b):
    F = fc_W.shape[1]
    C = cat.shape[1]
    return pl.pallas_call(
        _final_body,
        in_specs=[pl.BlockSpec(cat.shape, None),
                  pl.BlockSpec((C, F), None),
                  pl.BlockSpec((1, F), None),
                  pl.BlockSpec((1, F), None),
                  pl.BlockSpec((1, F), None),
                  pl.BlockSpec((F, 1), None),
                  pl.BlockSpec((1, 1), None)],
        out_specs=pl.BlockSpec((B, 1), None),
        out_shape=jax.ShapeDtypeStruct((B, 1), jnp.float32),
    )(cat, fc_W, fc_b.reshape(1, F), fc_gamma.reshape(1, F),
      fc_beta.reshape(1, F), out_W, out_b.reshape(1, 1))


def _gcn_predictor(x, edge_index, gids, p, halves):
    src, dst = edge_index[0], edge_index[1]
    lp1, lp2 = p['layers']
    hpre1 = _pre1_tc(x, lp1['W'], lp1['b'], halves)
    A1p = [_make_edge_agg(hw)(hp, src, dst) for hw, hp in zip(halves, hpre1)]
    h1, hpre2 = _mid_tc(A1p, x, lp1['Wr'], lp1['br'], lp1['gamma'],
                        lp1['beta'], lp2['W'], lp2['b'], halves)
    A2p = [_make_edge_agg(hw)(hp, src, dst) for hw, hp in zip(halves, hpre2)]
    h2, y = _post_tc(A2p, h1, lp2['Wr'], lp2['br'], lp2['gamma'],
                     lp2['beta'], p['atom_w'], p['atom_b'], halves)
    hsum = jax.ops.segment_sum(y, gids, num_segments=B)
    hmax = jax.ops.segment_max(h2, gids, num_segments=B)
    hmax = jnp.where(jnp.isfinite(hmax), hmax, 0.0)
    cat = jnp.concatenate([hsum, hmax], axis=1)
    return _pred_tc(cat, p['p_W1'], p['p_b1'], p['p_gamma'], p['p_beta'],
                    p['p_W2'], p['p_b2'])


def kernel(x_core, x_sub1, x_sub2, edge_index_core, edge_index_sub1,
           edge_index_sub2, graph_ids_core, graph_ids_sub1, graph_ids_sub2,
           params):
    core = _gcn_predictor(x_core, edge_index_core, graph_ids_core,
                          params['core'], (128, 128))
    sub1 = _gcn_predictor(x_sub1, edge_index_sub1, graph_ids_sub1,
                          params['sub'], (64,))
    sub2 = _gcn_predictor(x_sub2, edge_index_sub2, graph_ids_sub2,
                          params['sub'], (64,))
    cat = jnp.concatenate([core, sub1, sub2], axis=1)
    out = _final_tc(cat, params['fc_W'], params['fc_b'], params['fc_gamma'],
                    params['fc_beta'], params['out_W'], params['out_b'])
    return out.reshape(-1)


# speedup vs baseline: 3.8531x; 1.5061x over previous
import functools
import numpy as np
import jax
import jax.numpy as jnp
from jax import lax
from jax.experimental import pallas as pl
from jax.experimental.pallas import tpu as pltpu
from jax.experimental.pallas import tpu_sc as plsc

EPS = 1e-5
_SQ = float(np.sqrt(np.float32(1.0 + EPS)))
N = 10000
E = 160000
B = 256


def _dot(a, b):
    return jax.lax.dot_general(a, b, (((1,), (0,)), ((), ())),
                               preferred_element_type=jnp.float32)


_NCH = 39


def _edge_agg_body(F, v_hbm, src_hbm, dst_hbm, out_hbm,
                   stg0, stg1, sidx, didx, agg, sem0, sem1):
    zv = stg0
    RPT = 632
    LROWS = N - 15 * RPT
    c = lax.axis_index("c")
    s = lax.axis_index("s")
    w = c * 16 + s
    row0 = w * _NCH + jnp.minimum(w, 2)

    def zrow(r, carry):
        for j in range(F // 16):
            zv[r, pl.ds(j * 16, 16)] = jnp.zeros((16,), jnp.float32)
        return carry
    lax.fori_loop(0, 128, zrow, 0)

    @pl.when(w < 2)
    def _():
        pltpu.sync_copy(src_hbm.at[pl.ds(row0, _NCH + 1)], sidx)
        pltpu.sync_copy(dst_hbm.at[pl.ds(row0, _NCH + 1)], didx)

    @pl.when(w >= 2)
    def _():
        pltpu.sync_copy(src_hbm.at[pl.ds(row0, _NCH)],
                        sidx.at[pl.ds(0, _NCH)])
        pltpu.sync_copy(dst_hbm.at[pl.ds(row0, _NCH)],
                        didx.at[pl.ds(0, _NCH)])

    r0 = s * RPT

    def zero_rows(rows):
        nzc = rows // 128
        for j in range(nzc):
            pltpu.sync_copy(zv, agg.at[pl.ds(r0 + j * 128, 128)])
        rem = rows - nzc * 128
        if rem:
            pltpu.sync_copy(zv.at[pl.ds(0, rem)],
                            agg.at[pl.ds(r0 + nzc * 128, rem)])

    @pl.when(s < 15)
    def _():
        zero_rows(RPT)

    @pl.when(s == 15)
    def _():
        zero_rows(LROWS)
    plsc.subcore_barrier()

    bufs = (stg0, stg1)
    sems = (sem0, sem1)
    pend = [None] * (_NCH + 1)
    pend[0] = pltpu.async_copy(v_hbm.at[sidx.at[0]], stg0, sem0)
    for ci in range(_NCH):
        if ci + 1 < _NCH:
            pend[ci + 1] = pltpu.async_copy(
                v_hbm.at[sidx.at[ci + 1]], bufs[(ci + 1) % 2],
                sems[(ci + 1) % 2])
        pend[ci].wait()
        pltpu.sync_copy(bufs[ci % 2], agg.at[didx.at[ci]], add=True)

    @pl.when(w < 2)
    def _():
        pltpu.async_copy(v_hbm.at[sidx.at[_NCH]], stg0, sem0).wait()
        pltpu.sync_copy(stg0, agg.at[didx.at[_NCH]], add=True)
    plsc.subcore_barrier()

    @pl.when(s < 15)
    def _():
        pltpu.sync_copy(agg.at[pl.ds(r0, RPT)], out_hbm.at[c, pl.ds(r0, RPT)])

    @pl.when(s == 15)
    def _():
        pltpu.sync_copy(agg.at[pl.ds(r0, LROWS)],
                        out_hbm.at[c, pl.ds(r0, LROWS)])


@functools.lru_cache(maxsize=None)
def _make_edge_agg(F):
    mesh = plsc.VectorSubcoreMesh(core_axis_name="c", subcore_axis_name="s")
    return pl.kernel(
        functools.partial(_edge_agg_body, F),
        out_type=jax.ShapeDtypeStruct((2, N, F), jnp.float32),
        mesh=mesh,
        compiler_params=pltpu.CompilerParams(use_tc_tiling_on_sc=False),
        scratch_types=[
            pltpu.VMEM((128, F), jnp.float32),
            pltpu.VMEM((128, F), jnp.float32),
            pltpu.VMEM((_NCH + 1, 128), jnp.int32),
            pltpu.VMEM((_NCH + 1, 128), jnp.int32),
            pltpu.VMEM_SHARED((N, F), jnp.float32),
            pltpu.SemaphoreType.DMA,
            pltpu.SemaphoreType.DMA,
        ])


def _pre1_body(halves, x_ref, W_ref, b_ref, *o_refs):
    h = _dot(x_ref[...], W_ref[...]) + b_ref[...]
    off = 0
    for i, hw in enumerate(halves):
        o_refs[i][...] = h[:, off:off + hw]
        off += hw


def _pre1_tc(x, W, b, halves, blk=2000):
    H = W.shape[1]
    grid = N // blk
    row = lambda i: (i, 0)
    zero = lambda i: (0, 0)
    return pl.pallas_call(
        functools.partial(_pre1_body, tuple(halves)),
        grid=(grid,),
        in_specs=[pl.BlockSpec((blk, 32), row),
                  pl.BlockSpec((32, H), zero),
                  pl.BlockSpec((1, H), zero)],
        out_specs=[pl.BlockSpec((blk, hw), row) for hw in halves],
        out_shape=[jax.ShapeDtypeStruct((N, hw), jnp.float32)
                   for hw in halves],
    )(x, W, b.reshape(1, H))


def _mid_body(halves, nh, *refs):
    a_refs = refs[:nh]
    (x_ref, Wr_ref, br_ref, g_ref, be_ref, W2_ref, b2_ref) = refs[nh:nh + 7]
    o_refs = refs[nh + 7:]
    agg = jnp.concatenate([a[0] + a[1] for a in a_refs], axis=1) \
        if nh > 1 else a_refs[0][0] + a_refs[0][1]
    res = _dot(x_ref[...], Wr_ref[...]) + br_ref[...]
    new = jnp.maximum(agg, 0.0) + jnp.maximum(res, 0.0)
    h1 = g_ref[...] * new / _SQ + be_ref[...]
    o_refs[0][...] = h1
    hpre2 = _dot(h1, W2_ref[...]) + b2_ref[...]
    off = 0
    for i, hw in enumerate(halves):
        o_refs[1 + i][...] = hpre2[:, off:off + hw]
        off += hw


def _mid_tc(A1p, x, Wr, br, gamma, beta, W2, b2, halves, blk=2000):
    H = Wr.shape[1]
    grid = N // blk
    row = lambda i: (i, 0)
    row3 = lambda i: (0, i, 0)
    zero = lambda i: (0, 0)
    nh = len(halves)
    outs = pl.pallas_call(
        functools.partial(_mid_body, tuple(halves), nh),
        grid=(grid,),
        in_specs=[pl.BlockSpec((2, blk, hw), row3) for hw in halves]
        + [pl.BlockSpec((blk, 32), row),
           pl.BlockSpec((32, H), zero),
           pl.BlockSpec((1, H), zero),
           pl.BlockSpec((1, H), zero),
           pl.BlockSpec((1, H), zero),
           pl.BlockSpec((H, H), zero),
           pl.BlockSpec((1, H), zero)],
        out_specs=[pl.BlockSpec((blk, H), row)]
        + [pl.BlockSpec((blk, hw), row) for hw in halves],
        out_shape=[jax.ShapeDtypeStruct((N, H), jnp.float32)]
        + [jax.ShapeDtypeStruct((N, hw), jnp.float32) for hw in halves],
    )(*A1p, x, Wr, br.reshape(1, H), gamma.reshape(1, H), beta.reshape(1, H),
      W2, b2.reshape(1, H))
    return outs[0], outs[1:]


def _post_body(nh, *refs):
    a_refs = refs[:nh]
    (h1_ref, Wr_ref, br_ref, g_ref, be_ref, aw_ref, ab_ref) = refs[nh:nh + 7]
    h2_ref, y_ref = refs[nh + 7:]
    agg = jnp.concatenate([a[0] + a[1] for a in a_refs], axis=1) \
        if nh > 1 else a_refs[0][0] + a_refs[0][1]
    res = _dot(h1_ref[...], Wr_ref[...]) + br_ref[...]
    new = jnp.maximum(agg, 0.0) + jnp.maximum(res, 0.0)
    h2 = g_ref[...] * new / _SQ + be_ref[...]
    h2_ref[...] = h2
    z = _dot(h2, aw_ref[...]) + ab_ref[...]
    y_ref[...] = h2 * jax.nn.sigmoid(z)


def _post_tc(A2p, h1, Wr, br, gamma, beta, atom_w, atom_b, halves, blk=2000):
    H = Wr.shape[1]
    grid = N // blk
    row = lambda i: (i, 0)
    row3 = lambda i: (0, i, 0)
    zero = lambda i: (0, 0)
    nh = len(halves)
    return pl.pallas_call(
        functools.partial(_post_body, nh),
        grid=(grid,),
        in_specs=[pl.BlockSpec((2, blk, hw), row3) for hw in halves]
        + [pl.BlockSpec((blk, H), row),
           pl.BlockSpec((H, H), zero),
           pl.BlockSpec((1, H), zero),
           pl.BlockSpec((1, H), zero),
           pl.BlockSpec((1, H), zero),
           pl.BlockSpec((H, 1), zero),
           pl.BlockSpec((1, 1), zero)],
        out_specs=[pl.BlockSpec((blk, H), row), pl.BlockSpec((blk, H), row)],
        out_shape=[jax.ShapeDtypeStruct((N, H), jnp.float32),
                   jax.ShapeDtypeStruct((N, H), jnp.float32)],
    )(*A2p, h1, Wr, br.reshape(1, H), gamma.reshape(1, H),
      beta.reshape(1, H), atom_w, atom_b.reshape(1, 1))


def _pred_body(cat_ref, W1_ref, b1_ref, g_ref, be_ref, W2_ref, b2_ref,
               o_ref):
    g = _dot(cat_ref[...], W1_ref[...]) + b1_ref[...]
    g = jnp.maximum(g, 0.0)
    g = g_ref[...] * g / _SQ + be_ref[...]
    o_ref[...] = _dot(g, W2_ref[...]) + b2_ref[...]


def _pred_tc(cat, p_W1, p_b1, p_gamma, p_beta, p_W2, p_b2):
    H2 = cat.shape[1]
    P = p_W1.shape[1]
    T = p_W2.shape[1]
    return pl.pallas_call(
        _pred_body,
        in_specs=[pl.BlockSpec(cat.shape, None),
                  pl.BlockSpec((H2, P), None),
                  pl.BlockSpec((1, P), None),
                  pl.BlockSpec((1, P), None),
                  pl.BlockSpec((1, P), None),
                  pl.BlockSpec((P, T), None),
                  pl.BlockSpec((1, T), None)],
        out_specs=pl.BlockSpec((B, T), None),
        out_shape=jax.ShapeDtypeStruct((B, T), jnp.float32),
    )(cat, p_W1, p_b1.reshape(1, P), p_gamma.reshape(1, P),
      p_beta.reshape(1, P), p_W2, p_b2.reshape(1, T))


def _final_body(o_ref_in, W_ref, b_ref, g_ref, be_ref, ow_ref, ob_ref,
                o_ref):
    h = _dot(o_ref_in[...], W_ref[...]) + b_ref[...]
    h = g_ref[...] * h / _SQ + be_ref[...]
    h = jnp.maximum(h, 0.0)
    z = _dot(h, ow_ref[...]) + ob_ref[...]
    o_ref[...] = jax.nn.sigmoid(z)


def _final_tc(cat, fc_W, fc_b, fc_gamma, fc_beta, out_W, out_b):
    F = fc_W.shape[1]
    C = cat.shape[1]
    return pl.pallas_call(
        _final_body,
        in_specs=[pl.BlockSpec(cat.shape, None),
                  pl.BlockSpec((C, F), None),
                  pl.BlockSpec((1, F), None),
                  pl.BlockSpec((1, F), None),
                  pl.BlockSpec((1, F), None),
                  pl.BlockSpec((F, 1), None),
                  pl.BlockSpec((1, 1), None)],
        out_specs=pl.BlockSpec((B, 1), None),
        out_shape=jax.ShapeDtypeStruct((B, 1), jnp.float32),
    )(cat, fc_W, fc_b.reshape(1, F), fc_gamma.reshape(1, F),
      fc_beta.reshape(1, F), out_W, out_b.reshape(1, 1))


def _gcn_predictor(x, edge_index, gids, p, halves):
    src = edge_index[0].reshape(E // 128, 128)
    dst = edge_index[1].reshape(E // 128, 128)
    lp1, lp2 = p['layers']
    hpre1 = _pre1_tc(x, lp1['W'], lp1['b'], halves)
    A1p = [_make_edge_agg(hw)(hp, src, dst) for hw, hp in zip(halves, hpre1)]
    h1, hpre2 = _mid_tc(A1p, x, lp1['Wr'], lp1['br'], lp1['gamma'],
                        lp1['beta'], lp2['W'], lp2['b'], halves)
    A2p = [_make_edge_agg(hw)(hp, src, dst) for hw, hp in zip(halves, hpre2)]
    h2, y = _post_tc(A2p, h1, lp2['Wr'], lp2['br'], lp2['gamma'],
                     lp2['beta'], p['atom_w'], p['atom_b'], halves)
    hsum = jax.ops.segment_sum(y, gids, num_segments=B)
    hmax = jax.ops.segment_max(h2, gids, num_segments=B)
    hmax = jnp.where(jnp.isfinite(hmax), hmax, 0.0)
    cat = jnp.concatenate([hsum, hmax], axis=1)
    return _pred_tc(cat, p['p_W1'], p['p_b1'], p['p_gamma'], p['p_beta'],
                    p['p_W2'], p['p_b2'])


def kernel(x_core, x_sub1, x_sub2, edge_index_core, edge_index_sub1,
           edge_index_sub2, graph_ids_core, graph_ids_sub1, graph_ids_sub2,
           params):
    core = _gcn_predictor(x_core, edge_index_core, graph_ids_core,
                          params['core'], (128, 128))
    sub1 = _gcn_predictor(x_sub1, edge_index_sub1, graph_ids_sub1,
                          params['sub'], (64,))
    sub2 = _gcn_predictor(x_sub2, edge_index_sub2, graph_ids_sub2,
                          params['sub'], (64,))
    cat = jnp.concatenate([core, sub1, sub2], axis=1)
    out = _final_tc(cat, params['fc_W'], params['fc_b'], params['fc_gamma'],
                    params['fc_beta'], params['out_W'], params['out_b'])
    return out.reshape(-1)


# speedup vs baseline: 5.8903x; 1.5287x over previous
import functools
import numpy as np
import jax
import jax.numpy as jnp
from jax import lax
from jax.experimental import pallas as pl
from jax.experimental.pallas import tpu as pltpu
from jax.experimental.pallas import tpu_sc as plsc

EPS = 1e-5
_SQ = float(np.sqrt(np.float32(1.0 + EPS)))
N = 10000
E = 160000
B = 256


def _dot(a, b):
    return jax.lax.dot_general(a, b, (((1,), (0,)), ((), ())),
                               preferred_element_type=jnp.float32)


_NCH = 39


def _edge_agg_body(F, v_hbm, src_hbm, dst_hbm, out_hbm,
                   stg0, stg1, sidx, didx, agg, sem0, sem1):
    zv = stg0
    RPT = 632
    LROWS = N - 15 * RPT
    c = lax.axis_index("c")
    s = lax.axis_index("s")
    w = c * 16 + s
    row0 = w * _NCH + jnp.minimum(w, 2)

    def zrow(r, carry):
        for j in range(F // 16):
            zv[r, pl.ds(j * 16, 16)] = jnp.zeros((16,), jnp.float32)
        return carry
    lax.fori_loop(0, 128, zrow, 0)

    @pl.when(w < 2)
    def _():
        pltpu.sync_copy(src_hbm.at[pl.ds(row0, _NCH + 1)], sidx)
        pltpu.sync_copy(dst_hbm.at[pl.ds(row0, _NCH + 1)], didx)

    @pl.when(w >= 2)
    def _():
        pltpu.sync_copy(src_hbm.at[pl.ds(row0, _NCH)],
                        sidx.at[pl.ds(0, _NCH)])
        pltpu.sync_copy(dst_hbm.at[pl.ds(row0, _NCH)],
                        didx.at[pl.ds(0, _NCH)])

    r0 = s * RPT

    def zero_rows(rows):
        nzc = rows // 128
        for j in range(nzc):
            pltpu.sync_copy(zv, agg.at[pl.ds(r0 + j * 128, 128)])
        rem = rows - nzc * 128
        if rem:
            pltpu.sync_copy(zv.at[pl.ds(0, rem)],
                            agg.at[pl.ds(r0 + nzc * 128, rem)])

    @pl.when(s < 15)
    def _():
        zero_rows(RPT)

    @pl.when(s == 15)
    def _():
        zero_rows(LROWS)
    plsc.subcore_barrier()

    bufs = (stg0, stg1)
    sems = (sem0, sem1)
    pend = [None] * (_NCH + 1)
    pend[0] = pltpu.async_copy(v_hbm.at[sidx.at[0]], stg0, sem0)
    for ci in range(_NCH):
        if ci + 1 < _NCH:
            pend[ci + 1] = pltpu.async_copy(
                v_hbm.at[sidx.at[ci + 1]], bufs[(ci + 1) % 2],
                sems[(ci + 1) % 2])
        pend[ci].wait()
        pltpu.sync_copy(bufs[ci % 2], agg.at[didx.at[ci]], add=True)

    @pl.when(w < 2)
    def _():
        pltpu.async_copy(v_hbm.at[sidx.at[_NCH]], stg0, sem0).wait()
        pltpu.sync_copy(stg0, agg.at[didx.at[_NCH]], add=True)
    plsc.subcore_barrier()

    @pl.when(s < 15)
    def _():
        pltpu.sync_copy(agg.at[pl.ds(r0, RPT)], out_hbm.at[c, pl.ds(r0, RPT)])

    @pl.when(s == 15)
    def _():
        pltpu.sync_copy(agg.at[pl.ds(r0, LROWS)],
                        out_hbm.at[c, pl.ds(r0, LROWS)])


@functools.lru_cache(maxsize=None)
def _make_edge_agg(F):
    mesh = plsc.VectorSubcoreMesh(core_axis_name="c", subcore_axis_name="s")
    return pl.kernel(
        functools.partial(_edge_agg_body, F),
        out_type=jax.ShapeDtypeStruct((2, N, F), jnp.float32),
        mesh=mesh,
        compiler_params=pltpu.CompilerParams(use_tc_tiling_on_sc=False),
        scratch_types=[
            pltpu.VMEM((128, F), jnp.float32),
            pltpu.VMEM((128, F), jnp.float32),
            pltpu.VMEM((_NCH + 1, 128), jnp.int32),
            pltpu.VMEM((_NCH + 1, 128), jnp.int32),
            pltpu.VMEM_SHARED((N, F), jnp.float32),
            pltpu.SemaphoreType.DMA,
            pltpu.SemaphoreType.DMA,
        ])


_RCH = 80
_NRCH = N // _RCH


def _readout_body(F, h_hbm, y_hbm, gid_hbm, osum_hbm, omax_hbm,
                  hstg, ystg, gbuf, acc, ssum, sem):
    c = lax.axis_index("c")
    s = lax.axis_index("s")
    w = c * 16 + s
    NEG = jnp.full((16,), -jnp.inf, jnp.float32)

    def ninit(r, carry):
        for j in range(F // 16):
            acc[r, pl.ds(j * 16, 16)] = NEG
        return carry
    lax.fori_loop(0, B, ninit, 0)

    def zrow(r, carry):
        for j in range(F // 16):
            hstg[r, pl.ds(j * 16, 16)] = jnp.zeros((16,), jnp.float32)
        return carry
    lax.fori_loop(0, 16, zrow, 0)
    pltpu.sync_copy(hstg.at[pl.ds(0, 16)], ssum.at[pl.ds(s * 16, 16)])
    plsc.subcore_barrier()

    def do_chunk(k, hb, yb):
        ci = w + 32 * k
        pltpu.sync_copy(gid_hbm.at[pl.ds(ci, 1)], gbuf.at[pl.ds(k, 1)])
        pltpu.async_copy(h_hbm.at[pl.ds(ci * _RCH, _RCH)], hb, sem).wait()
        pltpu.sync_copy(y_hbm.at[pl.ds(ci * _RCH, _RCH)], yb)
        pltpu.sync_copy(yb, ssum.at[gbuf.at[k]], add=True)

        def row16(r16, carry):
            gvec = gbuf[k, pl.ds(r16 * 16, 16)]
            for lane in range(16):
                g = gvec[lane]
                r = r16 * 16 + lane
                for j in range(F // 16):
                    sl = pl.ds(j * 16, 16)
                    acc[g, sl] = jnp.maximum(acc[g, sl], hb[r, sl])
            return carry
        lax.fori_loop(0, _RCH // 16, row16, 0)

    for k in range(3):
        do_chunk(k, hstg, ystg)

    @pl.when(w + 96 < _NRCH)
    def _():
        do_chunk(3, hstg, ystg)
    plsc.subcore_barrier()

    pltpu.sync_copy(ssum.at[pl.ds(s * 16, 16)], osum_hbm.at[c, pl.ds(s * 16, 16)])
    pltpu.sync_copy(acc, omax_hbm.at[w])


@functools.lru_cache(maxsize=None)
def _make_readout(F):
    mesh = plsc.VectorSubcoreMesh(core_axis_name="c", subcore_axis_name="s")
    return pl.kernel(
        functools.partial(_readout_body, F),
        out_type=(jax.ShapeDtypeStruct((2, B, F), jnp.float32),
                  jax.ShapeDtypeStruct((32, B, F), jnp.float32)),
        mesh=mesh,
        compiler_params=pltpu.CompilerParams(use_tc_tiling_on_sc=False),
        scratch_types=[
            pltpu.VMEM((_RCH, F), jnp.float32),
            pltpu.VMEM((_RCH, F), jnp.float32),
            pltpu.VMEM((4, _RCH), jnp.int32),
            pltpu.VMEM((B, F), jnp.float32),
            pltpu.VMEM_SHARED((B, F), jnp.float32),
            pltpu.SemaphoreType.DMA,
        ])


def _pre1_body(halves, x_ref, W_ref, b_ref, *o_refs):
    h = _dot(x_ref[...], W_ref[...]) + b_ref[...]
    off = 0
    for i, hw in enumerate(halves):
        o_refs[i][...] = h[:, off:off + hw]
        off += hw


def _pre1_tc(x, W, b, halves, blk=2000):
    H = W.shape[1]
    grid = N // blk
    row = lambda i: (i, 0)
    zero = lambda i: (0, 0)
    return pl.pallas_call(
        functools.partial(_pre1_body, tuple(halves)),
        grid=(grid,),
        in_specs=[pl.BlockSpec((blk, 32), row),
                  pl.BlockSpec((32, H), zero),
                  pl.BlockSpec((1, H), zero)],
        out_specs=[pl.BlockSpec((blk, hw), row) for hw in halves],
        out_shape=[jax.ShapeDtypeStruct((N, hw), jnp.float32)
                   for hw in halves],
    )(x, W, b.reshape(1, H))


def _mid_body(halves, nh, *refs):
    a_refs = refs[:nh]
    (x_ref, Wr_ref, br_ref, g_ref, be_ref, W2_ref, b2_ref) = refs[nh:nh + 7]
    o_refs = refs[nh + 7:]
    agg = jnp.concatenate([a[0] + a[1] for a in a_refs], axis=1) \
        if nh > 1 else a_refs[0][0] + a_refs[0][1]
    res = _dot(x_ref[...], Wr_ref[...]) + br_ref[...]
    new = jnp.maximum(agg, 0.0) + jnp.maximum(res, 0.0)
    h1 = g_ref[...] * new / _SQ + be_ref[...]
    o_refs[0][...] = h1
    hpre2 = _dot(h1, W2_ref[...]) + b2_ref[...]
    off = 0
    for i, hw in enumerate(halves):
        o_refs[1 + i][...] = hpre2[:, off:off + hw]
        off += hw


def _mid_tc(A1p, x, Wr, br, gamma, beta, W2, b2, halves, blk=2000):
    H = Wr.shape[1]
    grid = N // blk
    row = lambda i: (i, 0)
    row3 = lambda i: (0, i, 0)
    zero = lambda i: (0, 0)
    nh = len(halves)
    outs = pl.pallas_call(
        functools.partial(_mid_body, tuple(halves), nh),
        grid=(grid,),
        in_specs=[pl.BlockSpec((2, blk, hw), row3) for hw in halves]
        + [pl.BlockSpec((blk, 32), row),
           pl.BlockSpec((32, H), zero),
           pl.BlockSpec((1, H), zero),
           pl.BlockSpec((1, H), zero),
           pl.BlockSpec((1, H), zero),
           pl.BlockSpec((H, H), zero),
           pl.BlockSpec((1, H), zero)],
        out_specs=[pl.BlockSpec((blk, H), row)]
        + [pl.BlockSpec((blk, hw), row) for hw in halves],
        out_shape=[jax.ShapeDtypeStruct((N, H), jnp.float32)]
        + [jax.ShapeDtypeStruct((N, hw), jnp.float32) for hw in halves],
    )(*A1p, x, Wr, br.reshape(1, H), gamma.reshape(1, H), beta.reshape(1, H),
      W2, b2.reshape(1, H))
    return outs[0], outs[1:]


def _post_body(halves, nh, *refs):
    a_refs = refs[:nh]
    (h1_ref, Wr_ref, br_ref, g_ref, be_ref, aw_ref, ab_ref) = refs[nh:nh + 7]
    o_refs = refs[nh + 7:]
    agg = jnp.concatenate([a[0] + a[1] for a in a_refs], axis=1) \
        if nh > 1 else a_refs[0][0] + a_refs[0][1]
    res = _dot(h1_ref[...], Wr_ref[...]) + br_ref[...]
    new = jnp.maximum(agg, 0.0) + jnp.maximum(res, 0.0)
    h2 = g_ref[...] * new / _SQ + be_ref[...]
    z = _dot(h2, aw_ref[...]) + ab_ref[...]
    y = h2 * jax.nn.sigmoid(z)
    off = 0
    for i, hw in enumerate(halves):
        o_refs[i][...] = h2[:, off:off + hw]
        o_refs[nh + i][...] = y[:, off:off + hw]
        off += hw


def _post_tc(A2p, h1, Wr, br, gamma, beta, atom_w, atom_b, halves, blk=2000):
    H = Wr.shape[1]
    grid = N // blk
    row = lambda i: (i, 0)
    row3 = lambda i: (0, i, 0)
    zero = lambda i: (0, 0)
    nh = len(halves)
    outs = pl.pallas_call(
        functools.partial(_post_body, tuple(halves), nh),
        grid=(grid,),
        in_specs=[pl.BlockSpec((2, blk, hw), row3) for hw in halves]
        + [pl.BlockSpec((blk, H), row),
           pl.BlockSpec((H, H), zero),
           pl.BlockSpec((1, H), zero),
           pl.BlockSpec((1, H), zero),
           pl.BlockSpec((1, H), zero),
           pl.BlockSpec((H, 1), zero),
           pl.BlockSpec((1, 1), zero)],
        out_specs=[pl.BlockSpec((blk, hw), row) for hw in halves] * 2,
        out_shape=[jax.ShapeDtypeStruct((N, hw), jnp.float32)
                   for hw in halves] * 2,
    )(*A2p, h1, Wr, br.reshape(1, H), gamma.reshape(1, H),
      beta.reshape(1, H), atom_w, atom_b.reshape(1, 1))
    return outs[:nh], outs[nh:]


def _pred_body(nh, H, *refs):
    s_refs = refs[:nh]
    m_refs = refs[nh:2 * nh]
    (W1_ref, b1_ref, g_ref, be_ref, W2_ref, b2_ref, o_ref) = refs[2 * nh:]
    hsum = jnp.concatenate([p[0] + p[1] for p in s_refs], axis=1) \
        if nh > 1 else s_refs[0][0] + s_refs[0][1]
    hmax = jnp.concatenate([jnp.max(p[...], axis=0) for p in m_refs],
                           axis=1) if nh > 1 else jnp.max(m_refs[0][...],
                                                          axis=0)
    hmax = jnp.where(jnp.isfinite(hmax), hmax, 0.0)
    W1 = W1_ref[...]
    g = _dot(hsum, W1[:H]) + _dot(hmax, W1[H:]) + b1_ref[...]
    g = jnp.maximum(g, 0.0)
    g = g_ref[...] * g / _SQ + be_ref[...]
    o_ref[...] = _dot(g, W2_ref[...]) + b2_ref[...]


def _pred_tc(sps, mps, p_W1, p_b1, p_gamma, p_beta, p_W2, p_b2):
    H2, P = p_W1.shape
    H = H2 // 2
    T = p_W2.shape[1]
    nh = len(sps)
    return pl.pallas_call(
        functools.partial(_pred_body, nh, H),
        in_specs=[pl.BlockSpec(p.shape, None) for p in sps]
        + [pl.BlockSpec(p.shape, None) for p in mps]
        + [pl.BlockSpec((H2, P), None),
           pl.BlockSpec((1, P), None),
           pl.BlockSpec((1, P), None),
           pl.BlockSpec((1, P), None),
           pl.BlockSpec((P, T), None),
           pl.BlockSpec((1, T), None)],
        out_specs=pl.BlockSpec((B, T), None),
        out_shape=jax.ShapeDtypeStruct((B, T), jnp.float32),
    )(*sps, *mps, p_W1, p_b1.reshape(1, P), p_gamma.reshape(1, P),
      p_beta.reshape(1, P), p_W2, p_b2.reshape(1, T))


def _final_body(o_ref_in, W_ref, b_ref, g_ref, be_ref, ow_ref, ob_ref,
                o_ref):
    h = _dot(o_ref_in[...], W_ref[...]) + b_ref[...]
    h = g_ref[...] * h / _SQ + be_ref[...]
    h = jnp.maximum(h, 0.0)
    z = _dot(h, ow_ref[...]) + ob_ref[...]
    o_ref[...] = jax.nn.sigmoid(z)


def _final_tc(cat, fc_W, fc_b, fc_gamma, fc_beta, out_W, out_b):
    F = fc_W.shape[1]
    C = cat.shape[1]
    return pl.pallas_call(
        _final_body,
        in_specs=[pl.BlockSpec(cat.shape, None),
                  pl.BlockSpec((C, F), None),
                  pl.BlockSpec((1, F), None),
                  pl.BlockSpec((1, F), None),
                  pl.BlockSpec((1, F), None),
                  pl.BlockSpec((F, 1), None),
                  pl.BlockSpec((1, 1), None)],
        out_specs=pl.BlockSpec((B, 1), None),
        out_shape=jax.ShapeDtypeStruct((B, 1), jnp.float32),
    )(cat, fc_W, fc_b.reshape(1, F), fc_gamma.reshape(1, F),
      fc_beta.reshape(1, F), out_W, out_b.reshape(1, 1))


def _gcn_predictor(x, edge_index, gids, p, halves):
    src = edge_index[0].reshape(E // 128, 128)
    dst = edge_index[1].reshape(E // 128, 128)
    lp1, lp2 = p['layers']
    hpre1 = _pre1_tc(x, lp1['W'], lp1['b'], halves)
    A1p = [_make_edge_agg(hw)(hp, src, dst) for hw, hp in zip(halves, hpre1)]
    h1, hpre2 = _mid_tc(A1p, x, lp1['Wr'], lp1['br'], lp1['gamma'],
                        lp1['beta'], lp2['W'], lp2['b'], halves)
    A2p = [_make_edge_agg(hw)(hp, src, dst) for hw, hp in zip(halves, hpre2)]
    h2s, ys = _post_tc(A2p, h1, lp2['Wr'], lp2['br'], lp2['gamma'],
                       lp2['beta'], p['atom_w'], p['atom_b'], halves)
    gid2 = gids.reshape(_NRCH, _RCH)
    sps, mps = [], []
    for hw, h2, y in zip(halves, h2s, ys):
        sp, mp = _make_readout(hw)(h2, y, gid2)
        sps.append(sp)
        mps.append(mp)
    return _pred_tc(sps, mps, p['p_W1'], p['p_b1'], p['p_gamma'],
                    p['p_beta'], p['p_W2'], p['p_b2'])


def kernel(x_core, x_sub1, x_sub2, edge_index_core, edge_index_sub1,
           edge_index_sub2, graph_ids_core, graph_ids_sub1, graph_ids_sub2,
           params):
    core = _gcn_predictor(x_core, edge_index_core, graph_ids_core,
                          params['core'], (128, 128))
    sub1 = _gcn_predictor(x_sub1, edge_index_sub1, graph_ids_sub1,
                          params['sub'], (64,))
    sub2 = _gcn_predictor(x_sub2, edge_index_sub2, graph_ids_sub2,
                          params['sub'], (64,))
    cat = jnp.concatenate([core, sub1, sub2], axis=1)
    out = _final_tc(cat, params['fc_W'], params['fc_b'], params['fc_gamma'],
                    params['fc_beta'], params['out_W'], params['out_b'])
    return out.reshape(-1)


# speedup vs baseline: 5.9078x; 1.0030x over previous
import functools
import numpy as np
import jax
import jax.numpy as jnp
from jax import lax
from jax.experimental import pallas as pl
from jax.experimental.pallas import tpu as pltpu
from jax.experimental.pallas import tpu_sc as plsc

EPS = 1e-5
_SQ = float(np.sqrt(np.float32(1.0 + EPS)))
N = 10000
E = 160000
B = 256


def _dot(a, b):
    return jax.lax.dot_general(a, b, (((1,), (0,)), ((), ())),
                               preferred_element_type=jnp.float32)


_NCH = 39


def _edge_agg_body(F, v_hbm, src_hbm, dst_hbm, out_hbm,
                   stg0, stg1, sidx, didx, agg, sem0, sem1, ssem0, ssem1):
    zv = stg0
    RPT = 632
    LROWS = N - 15 * RPT
    c = lax.axis_index("c")
    s = lax.axis_index("s")
    w = c * 16 + s
    row0 = w * _NCH + jnp.minimum(w, 2)

    def zrow(r, carry):
        for j in range(F // 16):
            zv[r, pl.ds(j * 16, 16)] = jnp.zeros((16,), jnp.float32)
        return carry
    lax.fori_loop(0, 128, zrow, 0)

    @pl.when(w < 2)
    def _():
        pltpu.sync_copy(src_hbm.at[pl.ds(row0, _NCH + 1)], sidx)
        pltpu.sync_copy(dst_hbm.at[pl.ds(row0, _NCH + 1)], didx)

    @pl.when(w >= 2)
    def _():
        pltpu.sync_copy(src_hbm.at[pl.ds(row0, _NCH)],
                        sidx.at[pl.ds(0, _NCH)])
        pltpu.sync_copy(dst_hbm.at[pl.ds(row0, _NCH)],
                        didx.at[pl.ds(0, _NCH)])

    r0 = s * RPT

    def zero_rows(rows):
        nzc = rows // 128
        for j in range(nzc):
            pltpu.sync_copy(zv, agg.at[pl.ds(r0 + j * 128, 128)])
        rem = rows - nzc * 128
        if rem:
            pltpu.sync_copy(zv.at[pl.ds(0, rem)],
                            agg.at[pl.ds(r0 + nzc * 128, rem)])

    @pl.when(s < 15)
    def _():
        zero_rows(RPT)

    @pl.when(s == 15)
    def _():
        zero_rows(LROWS)
    plsc.subcore_barrier()

    bufs = (stg0, stg1)
    sems = (sem0, sem1)
    ssems = (ssem0, ssem1)
    pend = [None] * (_NCH + 1)
    spend = [None] * (_NCH + 1)
    pend[0] = pltpu.async_copy(v_hbm.at[sidx.at[0]], stg0, sem0)
    for ci in range(_NCH):
        if ci + 1 < _NCH:
            if ci - 1 >= 0:
                spend[ci - 1].wait()
            pend[ci + 1] = pltpu.async_copy(
                v_hbm.at[sidx.at[ci + 1]], bufs[(ci + 1) % 2],
                sems[(ci + 1) % 2])
        pend[ci].wait()
        spend[ci] = pltpu.async_copy(bufs[ci % 2], agg.at[didx.at[ci]],
                                     ssems[ci % 2], add=True)
    spend[_NCH - 2].wait()
    spend[_NCH - 1].wait()

    @pl.when(w < 2)
    def _():
        pltpu.async_copy(v_hbm.at[sidx.at[_NCH]], stg0, sem0).wait()
        pltpu.sync_copy(stg0, agg.at[didx.at[_NCH]], add=True)
    plsc.subcore_barrier()

    @pl.when(s < 15)
    def _():
        pltpu.sync_copy(agg.at[pl.ds(r0, RPT)], out_hbm.at[c, pl.ds(r0, RPT)])

    @pl.when(s == 15)
    def _():
        pltpu.sync_copy(agg.at[pl.ds(r0, LROWS)],
                        out_hbm.at[c, pl.ds(r0, LROWS)])


@functools.lru_cache(maxsize=None)
def _make_edge_agg(F):
    mesh = plsc.VectorSubcoreMesh(core_axis_name="c", subcore_axis_name="s")
    return pl.kernel(
        functools.partial(_edge_agg_body, F),
        out_type=jax.ShapeDtypeStruct((2, N, F), jnp.float32),
        mesh=mesh,
        compiler_params=pltpu.CompilerParams(use_tc_tiling_on_sc=False),
        scratch_types=[
            pltpu.VMEM((128, F), jnp.float32),
            pltpu.VMEM((128, F), jnp.float32),
            pltpu.VMEM((_NCH + 1, 128), jnp.int32),
            pltpu.VMEM((_NCH + 1, 128), jnp.int32),
            pltpu.VMEM_SHARED((N, F), jnp.float32),
            pltpu.SemaphoreType.DMA,
            pltpu.SemaphoreType.DMA,
            pltpu.SemaphoreType.DMA,
            pltpu.SemaphoreType.DMA,
        ])


_RCH = 80
_NRCH = N // _RCH


def _readout_body(F, h_hbm, y_hbm, gid_hbm, osum_hbm, omax_hbm,
                  hstg, ystg, gbuf, acc, ssum, sem):
    c = lax.axis_index("c")
    s = lax.axis_index("s")
    w = c * 16 + s
    NEG = jnp.full((16,), -jnp.inf, jnp.float32)

    def ninit(r, carry):
        for j in range(F // 16):
            acc[r, pl.ds(j * 16, 16)] = NEG
        return carry
    lax.fori_loop(0, B, ninit, 0)

    def zrow(r, carry):
        for j in range(F // 16):
            hstg[r, pl.ds(j * 16, 16)] = jnp.zeros((16,), jnp.float32)
        return carry
    lax.fori_loop(0, 16, zrow, 0)
    pltpu.sync_copy(hstg.at[pl.ds(0, 16)], ssum.at[pl.ds(s * 16, 16)])
    plsc.subcore_barrier()

    def do_chunk(k, hb, yb):
        ci = w + 32 * k
        pltpu.sync_copy(gid_hbm.at[pl.ds(ci, 1)], gbuf.at[pl.ds(k, 1)])
        pltpu.async_copy(h_hbm.at[pl.ds(ci * _RCH, _RCH)], hb, sem).wait()
        pltpu.sync_copy(y_hbm.at[pl.ds(ci * _RCH, _RCH)], yb)
        pltpu.sync_copy(yb, ssum.at[gbuf.at[k]], add=True)

        def row16(r16, carry):
            gvec = gbuf[k, pl.ds(r16 * 16, 16)]
            for lane in range(16):
                g = gvec[lane]
                r = r16 * 16 + lane
                for j in range(F // 16):
                    sl = pl.ds(j * 16, 16)
                    acc[g, sl] = jnp.maximum(acc[g, sl], hb[r, sl])
            return carry
        lax.fori_loop(0, _RCH // 16, row16, 0)

    for k in range(3):
        do_chunk(k, hstg, ystg)

    @pl.when(w + 96 < _NRCH)
    def _():
        do_chunk(3, hstg, ystg)
    plsc.subcore_barrier()

    pltpu.sync_copy(ssum.at[pl.ds(s * 16, 16)], osum_hbm.at[c, pl.ds(s * 16, 16)])
    pltpu.sync_copy(acc, omax_hbm.at[w])


@functools.lru_cache(maxsize=None)
def _make_readout(F):
    mesh = plsc.VectorSubcoreMesh(core_axis_name="c", subcore_axis_name="s")
    return pl.kernel(
        functools.partial(_readout_body, F),
        out_type=(jax.ShapeDtypeStruct((2, B, F), jnp.float32),
                  jax.ShapeDtypeStruct((32, B, F), jnp.float32)),
        mesh=mesh,
        compiler_params=pltpu.CompilerParams(use_tc_tiling_on_sc=False),
        scratch_types=[
            pltpu.VMEM((_RCH, F), jnp.float32),
            pltpu.VMEM((_RCH, F), jnp.float32),
            pltpu.VMEM((4, _RCH), jnp.int32),
            pltpu.VMEM((B, F), jnp.float32),
            pltpu.VMEM_SHARED((B, F), jnp.float32),
            pltpu.SemaphoreType.DMA,
        ])


def _pre1_body(halves, x_ref, W_ref, b_ref, *o_refs):
    h = _dot(x_ref[...], W_ref[...]) + b_ref[...]
    off = 0
    for i, hw in enumerate(halves):
        o_refs[i][...] = h[:, off:off + hw]
        off += hw


def _pre1_tc(x, W, b, halves, blk=2000):
    H = W.shape[1]
    grid = N // blk
    row = lambda i: (i, 0)
    zero = lambda i: (0, 0)
    return pl.pallas_call(
        functools.partial(_pre1_body, tuple(halves)),
        grid=(grid,),
        in_specs=[pl.BlockSpec((blk, 32), row),
                  pl.BlockSpec((32, H), zero),
                  pl.BlockSpec((1, H), zero)],
        out_specs=[pl.BlockSpec((blk, hw), row) for hw in halves],
        out_shape=[jax.ShapeDtypeStruct((N, hw), jnp.float32)
                   for hw in halves],
    )(x, W, b.reshape(1, H))


def _mid_body(halves, nh, *refs):
    a_refs = refs[:nh]
    (x_ref, Wr_ref, br_ref, g_ref, be_ref, W2_ref, b2_ref) = refs[nh:nh + 7]
    o_refs = refs[nh + 7:]
    agg = jnp.concatenate([a[0] + a[1] for a in a_refs], axis=1) \
        if nh > 1 else a_refs[0][0] + a_refs[0][1]
    res = _dot(x_ref[...], Wr_ref[...]) + br_ref[...]
    new = jnp.maximum(agg, 0.0) + jnp.maximum(res, 0.0)
    h1 = g_ref[...] * new / _SQ + be_ref[...]
    o_refs[0][...] = h1
    hpre2 = _dot(h1, W2_ref[...]) + b2_ref[...]
    off = 0
    for i, hw in enumerate(halves):
        o_refs[1 + i][...] = hpre2[:, off:off + hw]
        off += hw


def _mid_tc(A1p, x, Wr, br, gamma, beta, W2, b2, halves, blk=2000):
    H = Wr.shape[1]
    grid = N // blk
    row = lambda i: (i, 0)
    row3 = lambda i: (0, i, 0)
    zero = lambda i: (0, 0)
    nh = len(halves)
    outs = pl.pallas_call(
        functools.partial(_mid_body, tuple(halves), nh),
        grid=(grid,),
        in_specs=[pl.BlockSpec((2, blk, hw), row3) for hw in halves]
        + [pl.BlockSpec((blk, 32), row),
           pl.BlockSpec((32, H), zero),
           pl.BlockSpec((1, H), zero),
           pl.BlockSpec((1, H), zero),
           pl.BlockSpec((1, H), zero),
           pl.BlockSpec((H, H), zero),
           pl.BlockSpec((1, H), zero)],
        out_specs=[pl.BlockSpec((blk, H), row)]
        + [pl.BlockSpec((blk, hw), row) for hw in halves],
        out_shape=[jax.ShapeDtypeStruct((N, H), jnp.float32)]
        + [jax.ShapeDtypeStruct((N, hw), jnp.float32) for hw in halves],
    )(*A1p, x, Wr, br.reshape(1, H), gamma.reshape(1, H), beta.reshape(1, H),
      W2, b2.reshape(1, H))
    return outs[0], outs[1:]


def _post_body(halves, nh, *refs):
    a_refs = refs[:nh]
    (h1_ref, Wr_ref, br_ref, g_ref, be_ref, aw_ref, ab_ref) = refs[nh:nh + 7]
    o_refs = refs[nh + 7:]
    agg = jnp.concatenate([a[0] + a[1] for a in a_refs], axis=1) \
        if nh > 1 else a_refs[0][0] + a_refs[0][1]
    res = _dot(h1_ref[...], Wr_ref[...]) + br_ref[...]
    new = jnp.maximum(agg, 0.0) + jnp.maximum(res, 0.0)
    h2 = g_ref[...] * new / _SQ + be_ref[...]
    z = _dot(h2, aw_ref[...]) + ab_ref[...]
    y = h2 * jax.nn.sigmoid(z)
    off = 0
    for i, hw in enumerate(halves):
        o_refs[i][...] = h2[:, off:off + hw]
        o_refs[nh + i][...] = y[:, off:off + hw]
        off += hw


def _post_tc(A2p, h1, Wr, br, gamma, beta, atom_w, atom_b, halves, blk=2000):
    H = Wr.shape[1]
    grid = N // blk
    row = lambda i: (i, 0)
    row3 = lambda i: (0, i, 0)
    zero = lambda i: (0, 0)
    nh = len(halves)
    outs = pl.pallas_call(
        functools.partial(_post_body, tuple(halves), nh),
        grid=(grid,),
        in_specs=[pl.BlockSpec((2, blk, hw), row3) for hw in halves]
        + [pl.BlockSpec((blk, H), row),
           pl.BlockSpec((H, H), zero),
           pl.BlockSpec((1, H), zero),
           pl.BlockSpec((1, H), zero),
           pl.BlockSpec((1, H), zero),
           pl.BlockSpec((H, 1), zero),
           pl.BlockSpec((1, 1), zero)],
        out_specs=[pl.BlockSpec((blk, hw), row) for hw in halves] * 2,
        out_shape=[jax.ShapeDtypeStruct((N, hw), jnp.float32)
                   for hw in halves] * 2,
    )(*A2p, h1, Wr, br.reshape(1, H), gamma.reshape(1, H),
      beta.reshape(1, H), atom_w, atom_b.reshape(1, 1))
    return outs[:nh], outs[nh:]


def _pred_body(nh, H, *refs):
    s_refs = refs[:nh]
    m_refs = refs[nh:2 * nh]
    (W1_ref, b1_ref, g_ref, be_ref, W2_ref, b2_ref, o_ref) = refs[2 * nh:]
    hsum = jnp.concatenate([p[0] + p[1] for p in s_refs], axis=1) \
        if nh > 1 else s_refs[0][0] + s_refs[0][1]
    hmax = jnp.concatenate([jnp.max(p[...], axis=0) for p in m_refs],
                           axis=1) if nh > 1 else jnp.max(m_refs[0][...],
                                                          axis=0)
    hmax = jnp.where(jnp.isfinite(hmax), hmax, 0.0)
    W1 = W1_ref[...]
    g = _dot(hsum, W1[:H]) + _dot(hmax, W1[H:]) + b1_ref[...]
    g = jnp.maximum(g, 0.0)
    g = g_ref[...] * g / _SQ + be_ref[...]
    o_ref[...] = _dot(g, W2_ref[...]) + b2_ref[...]


def _pred_tc(sps, mps, p_W1, p_b1, p_gamma, p_beta, p_W2, p_b2):
    H2, P = p_W1.shape
    H = H2 // 2
    T = p_W2.shape[1]
    nh = len(sps)
    return pl.pallas_call(
        functools.partial(_pred_body, nh, H),
        in_specs=[pl.BlockSpec(p.shape, None) for p in sps]
        + [pl.BlockSpec(p.shape, None) for p in mps]
        + [pl.BlockSpec((H2, P), None),
           pl.BlockSpec((1, P), None),
           pl.BlockSpec((1, P), None),
           pl.BlockSpec((1, P), None),
           pl.BlockSpec((P, T), None),
           pl.BlockSpec((1, T), None)],
        out_specs=pl.BlockSpec((B, T), None),
        out_shape=jax.ShapeDtypeStruct((B, T), jnp.float32),
    )(*sps, *mps, p_W1, p_b1.reshape(1, P), p_gamma.reshape(1, P),
      p_beta.reshape(1, P), p_W2, p_b2.reshape(1, T))


def _final_body(o_ref_in, W_ref, b_ref, g_ref, be_ref, ow_ref, ob_ref,
                o_ref):
    h = _dot(o_ref_in[...], W_ref[...]) + b_ref[...]
    h = g_ref[...] * h / _SQ + be_ref[...]
    h = jnp.maximum(h, 0.0)
    z = _dot(h, ow_ref[...]) + ob_ref[...]
    o_ref[...] = jax.nn.sigmoid(z)


def _final_tc(cat, fc_W, fc_b, fc_gamma, fc_beta, out_W, out_b):
    F = fc_W.shape[1]
    C = cat.shape[1]
    return pl.pallas_call(
        _final_body,
        in_specs=[pl.BlockSpec(cat.shape, None),
                  pl.BlockSpec((C, F), None),
                  pl.BlockSpec((1, F), None),
                  pl.BlockSpec((1, F), None),
                  pl.BlockSpec((1, F), None),
                  pl.BlockSpec((F, 1), None),
                  pl.BlockSpec((1, 1), None)],
        out_specs=pl.BlockSpec((B, 1), None),
        out_shape=jax.ShapeDtypeStruct((B, 1), jnp.float32),
    )(cat, fc_W, fc_b.reshape(1, F), fc_gamma.reshape(1, F),
      fc_beta.reshape(1, F), out_W, out_b.reshape(1, 1))


def _gcn_predictor(x, edge_index, gids, p, halves):
    src = edge_index[0].reshape(E // 128, 128)
    dst = edge_index[1].reshape(E // 128, 128)
    lp1, lp2 = p['layers']
    hpre1 = _pre1_tc(x, lp1['W'], lp1['b'], halves)
    A1p = [_make_edge_agg(hw)(hp, src, dst) for hw, hp in zip(halves, hpre1)]
    h1, hpre2 = _mid_tc(A1p, x, lp1['Wr'], lp1['br'], lp1['gamma'],
                        lp1['beta'], lp2['W'], lp2['b'], halves)
    A2p = [_make_edge_agg(hw)(hp, src, dst) for hw, hp in zip(halves, hpre2)]
    h2s, ys = _post_tc(A2p, h1, lp2['Wr'], lp2['br'], lp2['gamma'],
                       lp2['beta'], p['atom_w'], p['atom_b'], halves)
    gid2 = gids.reshape(_NRCH, _RCH)
    sps, mps = [], []
    for hw, h2, y in zip(halves, h2s, ys):
        sp, mp = _make_readout(hw)(h2, y, gid2)
        sps.append(sp)
        mps.append(mp)
    return _pred_tc(sps, mps, p['p_W1'], p['p_b1'], p['p_gamma'],
                    p['p_beta'], p['p_W2'], p['p_b2'])


def kernel(x_core, x_sub1, x_sub2, edge_index_core, edge_index_sub1,
           edge_index_sub2, graph_ids_core, graph_ids_sub1, graph_ids_sub2,
           params):
    core = _gcn_predictor(x_core, edge_index_core, graph_ids_core,
                          params['core'], (128, 128))
    sub1 = _gcn_predictor(x_sub1, edge_index_sub1, graph_ids_sub1,
                          params['sub'], (64,))
    sub2 = _gcn_predictor(x_sub2, edge_index_sub2, graph_ids_sub2,
                          params['sub'], (64,))
    cat = jnp.concatenate([core, sub1, sub2], axis=1)
    out = _final_tc(cat, params['fc_W'], params['fc_b'], params['fc_gamma'],
                    params['fc_beta'], params['out_W'], params['out_b'])
    return out.reshape(-1)
